# Initial kernel scaffold; baseline (speedup 1.0000x reference)
import jax
import jax.numpy as jnp
from jax.experimental import pallas as pl


def kernel(feat, segment_ids):
    raise NotImplementedError("write your pallas kernel here")


# speedup vs baseline: 1.9518x; 1.9518x over previous
import functools

import jax
import jax.numpy as jnp
from jax import lax
from jax.experimental import pallas as pl
from jax.experimental.pallas import tpu as pltpu
from jax.experimental.pallas import tpu_sc as plsc

N_ROWS = 50000
D = 256
G = 128
BLK = 64
NBLK = N_ROWS // BLK
TAIL = N_ROWS - NBLK * BLK
NC = 2
NS = 16
NW = NC * NS
BASE_BLKS = NBLK // NW
EXTRA_BLKS = NBLK - BASE_BLKS * NW
ROWS_PER_TILE = G // NS


def _sc_partials(feat, ids):
  mesh = plsc.VectorSubcoreMesh(core_axis_name="c", subcore_axis_name="s")

  @functools.partial(
      pl.kernel,
      out_type=jax.ShapeDtypeStruct((NC, G, D), jnp.float32),
      mesh=mesh,
      scratch_types=[
          pltpu.VMEM((BLK,), jnp.int32),
          pltpu.VMEM((BLK, D), jnp.float32),
          pltpu.VMEM((TAIL,), jnp.int32),
          pltpu.VMEM((TAIL, D), jnp.float32),
          pltpu.VMEM((G, D), jnp.float32),
          pltpu.VMEM((ROWS_PER_TILE, D), jnp.float32),
          pltpu.VMEM_SHARED((NS, G, D), jnp.float32),
      ],
  )
  def k(feat_hbm, ids_hbm, out_hbm, ids_v, rows_v, tids_v, trows_v, acc,
        tbuf, shared):
    c = lax.axis_index("c")
    s = lax.axis_index("s")
    w = s * NC + c

    zero = jnp.zeros((16,), jnp.float32)

    def zero_body(i, carry):
      r = i // (D // 16)
      col = (i % (D // 16)) * 16
      acc[r, pl.ds(col, 16)] = zero
      return carry

    lax.fori_loop(0, G * (D // 16), zero_body, 0)

    base = w * BASE_BLKS + jnp.minimum(w, EXTRA_BLKS)
    nblk = BASE_BLKS + jnp.where(w < EXTRA_BLKS, 1, 0)

    def accum_group(g2, idref, rowref):
      idv = idref[pl.ds(g2 * 16, 16)]
      for j in range(16):
        sj = idv[j]
        r = g2 * 16 + j
        for kc in range(D // 16):
          col = kc * 16
          acc[sj, pl.ds(col, 16)] = (
              acc[sj, pl.ds(col, 16)] + rowref[r, pl.ds(col, 16)])

    def body(g, carry):
      r0 = (base + g) * BLK
      pltpu.sync_copy(ids_hbm.at[pl.ds(r0, BLK)], ids_v)
      pltpu.sync_copy(feat_hbm.at[pl.ds(r0, BLK)], rows_v)

      def group_body(g2, carry2):
        accum_group(g2, ids_v, rows_v)
        return carry2

      lax.fori_loop(0, BLK // 16, group_body, 0)
      return carry

    lax.fori_loop(0, nblk, body, 0)

    @pl.when(w == NW - 1)
    def _():
      r0 = NBLK * BLK
      pltpu.sync_copy(ids_hbm.at[pl.ds(r0, TAIL)], tids_v)
      pltpu.sync_copy(feat_hbm.at[pl.ds(r0, TAIL)], trows_v)
      accum_group(0, tids_v, trows_v)

    pltpu.sync_copy(acc, shared.at[s])
    plsc.subcore_barrier()

    myrow = s * ROWS_PER_TILE

    pltpu.sync_copy(
        shared.at[0, pl.ds(myrow, ROWS_PER_TILE)],
        acc.at[pl.ds(myrow, ROWS_PER_TILE)])

    def red_body(t, carry):
      pltpu.sync_copy(shared.at[t, pl.ds(myrow, ROWS_PER_TILE)], tbuf)

      def add_body(i, carry2):
        r = i // (D // 16)
        col = (i % (D // 16)) * 16
        acc[myrow + r, pl.ds(col, 16)] = (
            acc[myrow + r, pl.ds(col, 16)] + tbuf[r, pl.ds(col, 16)])
        return carry2

      lax.fori_loop(0, ROWS_PER_TILE * (D // 16), add_body, 0)
      return carry

    lax.fori_loop(1, NS, red_body, 0)

    pltpu.sync_copy(
        acc.at[pl.ds(myrow, ROWS_PER_TILE)],
        out_hbm.at[c, pl.ds(myrow, ROWS_PER_TILE)],
    )

  return k(feat, ids)


def _combine(parts):
  def body(p_ref, o_ref):
    o_ref[...] = p_ref[0] + p_ref[1]

  return pl.pallas_call(
      body,
      out_shape=jax.ShapeDtypeStruct((G, D), jnp.float32),
  )(parts)


def kernel(feat, segment_ids):
  ids = segment_ids.astype(jnp.int32)
  parts = _sc_partials(feat, ids)
  return _combine(parts)


# speedup vs baseline: 4.1152x; 2.1085x over previous
import functools

import jax
import jax.numpy as jnp
from jax import lax
from jax.experimental import pallas as pl
from jax.experimental.pallas import tpu as pltpu
from jax.experimental.pallas import tpu_sc as plsc

N_ROWS = 50000
D = 256
G = 128
BLK = 64
NBLK = N_ROWS // BLK
TAIL = N_ROWS - NBLK * BLK
NC = 2
NS = 16
NW = NC * NS
BASE_BLKS = NBLK // NW
EXTRA_BLKS = NBLK - BASE_BLKS * NW
ROWS_PER_TILE = G // NS


def _sc_partials(feat, ids):
  mesh = plsc.VectorSubcoreMesh(core_axis_name="c", subcore_axis_name="s")

  @functools.partial(
      pl.kernel,
      out_type=jax.ShapeDtypeStruct((NC, G, D), jnp.float32),
      mesh=mesh,
      scratch_types=[
          pltpu.VMEM((2, BLK), jnp.int32),
          pltpu.VMEM((2, BLK, D), jnp.float32),
          pltpu.VMEM((TAIL,), jnp.int32),
          pltpu.VMEM((TAIL, D), jnp.float32),
          pltpu.VMEM((G, D), jnp.float32),
          pltpu.VMEM((ROWS_PER_TILE, D), jnp.float32),
          pltpu.VMEM_SHARED((NS, G, D), jnp.float32),
          pltpu.SemaphoreType.DMA,
          pltpu.SemaphoreType.DMA,
      ],
  )
  def k(feat_hbm, ids_hbm, out_hbm, ids2, rows2, tids_v, trows_v, acc,
        tbuf, shared, sem0, sem1):
    c = lax.axis_index("c")
    s = lax.axis_index("s")
    w = s * NC + c
    sems = (sem0, sem1)

    zero = jnp.zeros((16,), jnp.float32)

    def zero_body(i, carry):
      r = i // (D // 16)
      col = (i % (D // 16)) * 16
      acc[r, pl.ds(col, 16)] = zero
      return carry

    lax.fori_loop(0, G * (D // 16), zero_body, 0)

    base = w * BASE_BLKS + jnp.minimum(w, EXTRA_BLKS)
    nblk = BASE_BLKS + jnp.where(w < EXTRA_BLKS, 1, 0)

    def accum_group(gbase, idref, rowref):
      idv = idref[pl.ds(gbase, 16)]
      first = idv[0]
      last = idv[15]

      def fast():
        for kc in range(D // 16):
          col = kc * 16
          v = [rowref[gbase + j, pl.ds(col, 16)] for j in range(16)]
          while len(v) > 1:
            v = [v[i] + v[i + 1] for i in range(0, len(v), 2)]
          acc[first, pl.ds(col, 16)] = acc[first, pl.ds(col, 16)] + v[0]

      def slow():
        for j in range(16):
          sj = idv[j]
          for kc in range(D // 16):
            col = kc * 16
            acc[sj, pl.ds(col, 16)] = (
                acc[sj, pl.ds(col, 16)] + rowref[gbase + j, pl.ds(col, 16)])

      lax.cond(first == last, fast, slow)

    def start_blk(g, b):
      r0 = (base + g) * BLK
      pltpu.make_async_copy(
          ids_hbm.at[pl.ds(r0, BLK)], ids2.at[b], sems[b]).start()
      pltpu.make_async_copy(
          feat_hbm.at[pl.ds(r0, BLK)], rows2.at[b], sems[b]).start()

    def wait_blk(b):
      pltpu.make_async_copy(
          ids_hbm.at[pl.ds(0, BLK)], ids2.at[b], sems[b]).wait()
      pltpu.make_async_copy(
          feat_hbm.at[pl.ds(0, BLK)], rows2.at[b], sems[b]).wait()

    @pl.when(nblk > 0)
    def _():
      start_blk(0, 0)

    @pl.when(nblk > 1)
    def _():
      start_blk(1, 1)

    def pair_body(p, carry):
      for b in range(2):
        g = p * 2 + b

        @pl.when(g < nblk)
        def _():
          wait_blk(b)

          def group_body(g2, carry2):
            accum_group(g2 * 16, ids2.at[b], rows2.at[b])
            return carry2

          lax.fori_loop(0, BLK // 16, group_body, 0)

          @pl.when(g + 2 < nblk)
          def _():
            start_blk(g + 2, b)

      return carry

    lax.fori_loop(0, (nblk + 1) // 2, pair_body, 0)

    @pl.when(w == NW - 1)
    def _():
      r0 = NBLK * BLK
      pltpu.sync_copy(ids_hbm.at[pl.ds(r0, TAIL)], tids_v)
      pltpu.sync_copy(feat_hbm.at[pl.ds(r0, TAIL)], trows_v)
      accum_group(0, tids_v, trows_v)

    pltpu.sync_copy(acc, shared.at[s])
    plsc.subcore_barrier()

    myrow = s * ROWS_PER_TILE

    pltpu.sync_copy(
        shared.at[0, pl.ds(myrow, ROWS_PER_TILE)],
        acc.at[pl.ds(myrow, ROWS_PER_TILE)])

    def red_body(t, carry):
      pltpu.sync_copy(shared.at[t, pl.ds(myrow, ROWS_PER_TILE)], tbuf)

      def add_body(i, carry2):
        r = i // (D // 16)
        col = (i % (D // 16)) * 16
        acc[myrow + r, pl.ds(col, 16)] = (
            acc[myrow + r, pl.ds(col, 16)] + tbuf[r, pl.ds(col, 16)])
        return carry2

      lax.fori_loop(0, ROWS_PER_TILE * (D // 16), add_body, 0)
      return carry

    lax.fori_loop(1, NS, red_body, 0)

    pltpu.sync_copy(
        acc.at[pl.ds(myrow, ROWS_PER_TILE)],
        out_hbm.at[c, pl.ds(myrow, ROWS_PER_TILE)],
    )

  return k(feat, ids)


def _combine(parts):
  def body(p_ref, o_ref):
    o_ref[...] = p_ref[0] + p_ref[1]

  return pl.pallas_call(
      body,
      out_shape=jax.ShapeDtypeStruct((G, D), jnp.float32),
  )(parts)


def kernel(feat, segment_ids):
  ids = segment_ids.astype(jnp.int32)
  parts = _sc_partials(feat, ids)
  return _combine(parts)


# speedup vs baseline: 4.5880x; 1.1149x over previous
import functools

import jax
import jax.numpy as jnp
from jax import lax
from jax.experimental import pallas as pl
from jax.experimental.pallas import tpu as pltpu
from jax.experimental.pallas import tpu_sc as plsc

N_ROWS = 50000
D = 256
G = 128
BLK = 96
NBLK = N_ROWS // BLK
TAIL = N_ROWS - NBLK * BLK
NC = 2
NS = 16
NW = NC * NS
BASE_BLKS = NBLK // NW
EXTRA_BLKS = NBLK - BASE_BLKS * NW
ROWS_PER_TILE = G // NS


def _sc_partials(feat, ids):
  mesh = plsc.VectorSubcoreMesh(core_axis_name="c", subcore_axis_name="s")

  @functools.partial(
      pl.kernel,
      out_type=jax.ShapeDtypeStruct((NC, G, D), jnp.float32),
      mesh=mesh,
      scratch_types=[
          pltpu.VMEM((2, BLK), jnp.int32),
          pltpu.VMEM((2, BLK, D), jnp.float32),
          pltpu.VMEM((G, D), jnp.float32),
          pltpu.VMEM((ROWS_PER_TILE, D), jnp.float32),
          pltpu.VMEM_SHARED((NS, G, D), jnp.float32),
          pltpu.SemaphoreType.DMA,
          pltpu.SemaphoreType.DMA,
      ],
  )
  def k(feat_hbm, ids_hbm, out_hbm, ids2, rows2, acc, tbuf, shared, sem0,
        sem1):
    c = lax.axis_index("c")
    s = lax.axis_index("s")
    w = s * NC + c
    sems = (sem0, sem1)

    zero = jnp.zeros((16,), jnp.float32)

    def zero_body(r, carry):
      for kc in range(D // 16):
        acc[r, pl.ds(kc * 16, 16)] = zero
      return carry

    lax.fori_loop(0, G, zero_body, 0)

    base = w * BASE_BLKS + jnp.minimum(w, EXTRA_BLKS)
    nblk = BASE_BLKS + jnp.where(w < EXTRA_BLKS, 1, 0)

    def accum_group(gbase, idref, rowref):
      idv = idref[pl.ds(gbase, 16)]
      first = idv[0]
      last = idv[15]

      def fast():
        for kc in range(D // 16):
          col = kc * 16
          v = [rowref[gbase + j, pl.ds(col, 16)] for j in range(16)]
          while len(v) > 1:
            v = [v[i] + v[i + 1] for i in range(0, len(v), 2)]
          acc[first, pl.ds(col, 16)] = acc[first, pl.ds(col, 16)] + v[0]

      def slow():
        for j in range(16):
          sj = idv[j]
          for kc in range(D // 16):
            col = kc * 16
            acc[sj, pl.ds(col, 16)] = (
                acc[sj, pl.ds(col, 16)] + rowref[gbase + j, pl.ds(col, 16)])

      lax.cond(first == last, fast, slow)

    def start_blk(g, b):
      r0 = (base + g) * BLK
      pltpu.make_async_copy(
          ids_hbm.at[pl.ds(r0, BLK)], ids2.at[b], sems[b]).start()
      pltpu.make_async_copy(
          feat_hbm.at[pl.ds(r0, BLK)], rows2.at[b], sems[b]).start()

    def wait_blk(b):
      pltpu.make_async_copy(
          ids_hbm.at[pl.ds(0, BLK)], ids2.at[b], sems[b]).wait()
      pltpu.make_async_copy(
          feat_hbm.at[pl.ds(0, BLK)], rows2.at[b], sems[b]).wait()

    @pl.when(nblk > 0)
    def _():
      start_blk(0, 0)

    @pl.when(nblk > 1)
    def _():
      start_blk(1, 1)

    def pair_body(p, carry):
      for b in range(2):
        g = p * 2 + b

        @pl.when(g < nblk)
        def _():
          wait_blk(b)

          def group_body(g2, carry2):
            accum_group(g2 * 16, ids2.at[b], rows2.at[b])
            return carry2

          lax.fori_loop(0, BLK // 16, group_body, 0)

          @pl.when(g + 2 < nblk)
          def _():
            start_blk(g + 2, b)

      return carry

    lax.fori_loop(0, (nblk + 1) // 2, pair_body, 0)

    @pl.when(w == NW - 1)
    def _():
      r0 = NBLK * BLK
      pltpu.sync_copy(ids_hbm.at[pl.ds(r0, TAIL)], ids2.at[0, pl.ds(0, TAIL)])
      pltpu.sync_copy(feat_hbm.at[pl.ds(r0, TAIL)],
                      rows2.at[0, pl.ds(0, TAIL)])

      def tail_body(t, carry):
        accum_group(t * 16, ids2.at[0], rows2.at[0])
        return carry

      lax.fori_loop(0, TAIL // 16, tail_body, 0)

    pltpu.sync_copy(acc, shared.at[s])
    plsc.subcore_barrier()

    myrow = s * ROWS_PER_TILE

    pltpu.sync_copy(
        shared.at[0, pl.ds(myrow, ROWS_PER_TILE)],
        acc.at[pl.ds(myrow, ROWS_PER_TILE)])

    def red_body(t, carry):
      pltpu.sync_copy(shared.at[t, pl.ds(myrow, ROWS_PER_TILE)], tbuf)

      def add_body(i, carry2):
        r = i // (D // 16)
        col = (i % (D // 16)) * 16
        acc[myrow + r, pl.ds(col, 16)] = (
            acc[myrow + r, pl.ds(col, 16)] + tbuf[r, pl.ds(col, 16)])
        return carry2

      lax.fori_loop(0, ROWS_PER_TILE * (D // 16), add_body, 0)
      return carry

    lax.fori_loop(1, NS, red_body, 0)

    pltpu.sync_copy(
        acc.at[pl.ds(myrow, ROWS_PER_TILE)],
        out_hbm.at[c, pl.ds(myrow, ROWS_PER_TILE)],
    )

  return k(feat, ids)


def _combine(parts):
  def body(p_ref, o_ref):
    o_ref[...] = p_ref[0] + p_ref[1]

  return pl.pallas_call(
      body,
      out_shape=jax.ShapeDtypeStruct((G, D), jnp.float32),
  )(parts)


def kernel(feat, segment_ids):
  ids = segment_ids.astype(jnp.int32)
  parts = _sc_partials(feat, ids)
  return _combine(parts)


# speedup vs baseline: 5.5951x; 1.2195x over previous
import functools

import jax
import jax.numpy as jnp
from jax import lax
from jax.experimental import pallas as pl
from jax.experimental.pallas import tpu as pltpu
from jax.experimental.pallas import tpu_sc as plsc

N_ROWS = 50000
D = 256
G = 128
NC = 2
NS = 16
DC = D // NC
BLK = 256
NBLK = N_ROWS // BLK
TAIL = N_ROWS - NBLK * BLK
BASE_BLKS = NBLK // NS
EXTRA_BLKS = NBLK - BASE_BLKS * NS
ROWS_PER_TILE = G // NS
KC = DC // 16


def _sum_pool(feat, ids):
  mesh = plsc.VectorSubcoreMesh(core_axis_name="c", subcore_axis_name="s")

  @functools.partial(
      pl.kernel,
      out_type=jax.ShapeDtypeStruct((G, D), jnp.float32),
      mesh=mesh,
      scratch_types=[
          pltpu.VMEM((2, BLK), jnp.int32),
          pltpu.VMEM((2, BLK, DC), jnp.float32),
          pltpu.VMEM((G, DC), jnp.float32),
          pltpu.VMEM((ROWS_PER_TILE, DC), jnp.float32),
          pltpu.VMEM_SHARED((NS, G, DC), jnp.float32),
          pltpu.SemaphoreType.DMA,
          pltpu.SemaphoreType.DMA,
      ],
  )
  def k(feat_hbm, ids_hbm, out_hbm, ids2, rows2, acc, tbuf, shared, sem0,
        sem1):
    c = lax.axis_index("c")
    s = lax.axis_index("s")
    sems = (sem0, sem1)
    col0 = c * DC

    zero = jnp.zeros((16,), jnp.float32)

    def zero_body(r, carry):
      for kc in range(KC):
        acc[r, pl.ds(kc * 16, 16)] = zero
      return carry

    lax.fori_loop(0, G, zero_body, 0)

    base = s * BASE_BLKS + jnp.minimum(s, EXTRA_BLKS)
    nblk = BASE_BLKS + jnp.where(s < EXTRA_BLKS, 1, 0)

    def accum_group(gbase, idref, rowref):
      idv = idref[pl.ds(gbase, 16)]
      first = idv[0]
      last = idv[15]

      def fast():
        for kc in range(KC):
          col = kc * 16
          v = [rowref[gbase + j, pl.ds(col, 16)] for j in range(16)]
          while len(v) > 1:
            v = [v[i] + v[i + 1] for i in range(0, len(v), 2)]
          acc[first, pl.ds(col, 16)] = acc[first, pl.ds(col, 16)] + v[0]

      def slow():
        for j in range(16):
          sj = idv[j]
          for kc in range(KC):
            col = kc * 16
            acc[sj, pl.ds(col, 16)] = (
                acc[sj, pl.ds(col, 16)] + rowref[gbase + j, pl.ds(col, 16)])

      lax.cond(first == last, fast, slow)

    def start_blk(g, b):
      r0 = (base + g) * BLK
      pltpu.make_async_copy(
          ids_hbm.at[pl.ds(r0, BLK)], ids2.at[b], sems[b]).start()
      pltpu.make_async_copy(
          feat_hbm.at[pl.ds(r0, BLK), pl.ds(col0, DC)], rows2.at[b],
          sems[b]).start()

    def wait_blk(b):
      pltpu.make_async_copy(
          ids_hbm.at[pl.ds(0, BLK)], ids2.at[b], sems[b]).wait()
      pltpu.make_async_copy(
          feat_hbm.at[pl.ds(0, BLK), pl.ds(0, DC)], rows2.at[b],
          sems[b]).wait()

    @pl.when(nblk > 0)
    def _():
      start_blk(0, 0)

    @pl.when(nblk > 1)
    def _():
      start_blk(1, 1)

    def pair_body(p, carry):
      for b in range(2):
        g = p * 2 + b

        @pl.when(g < nblk)
        def _():
          wait_blk(b)

          def group_body(g2, carry2):
            accum_group(g2 * 16, ids2.at[b], rows2.at[b])
            return carry2

          lax.fori_loop(0, BLK // 16, group_body, 0)

          @pl.when(g + 2 < nblk)
          def _():
            start_blk(g + 2, b)

      return carry

    lax.fori_loop(0, (nblk + 1) // 2, pair_body, 0)

    @pl.when(s == NS - 1)
    def _():
      r0 = NBLK * BLK
      pltpu.sync_copy(ids_hbm.at[pl.ds(r0, TAIL)], ids2.at[0, pl.ds(0, TAIL)])
      pltpu.sync_copy(feat_hbm.at[pl.ds(r0, TAIL), pl.ds(col0, DC)],
                      rows2.at[0, pl.ds(0, TAIL)])

      def tail_body(t, carry):
        accum_group(t * 16, ids2.at[0], rows2.at[0])
        return carry

      lax.fori_loop(0, TAIL // 16, tail_body, 0)

    pltpu.sync_copy(acc, shared.at[s])
    plsc.subcore_barrier()

    myrow = s * ROWS_PER_TILE

    pltpu.sync_copy(
        shared.at[0, pl.ds(myrow, ROWS_PER_TILE)],
        acc.at[pl.ds(myrow, ROWS_PER_TILE)])

    def red_body(t, carry):
      pltpu.sync_copy(shared.at[t, pl.ds(myrow, ROWS_PER_TILE)], tbuf)
      for r in range(ROWS_PER_TILE):
        for kc in range(KC):
          col = kc * 16
          acc[myrow + r, pl.ds(col, 16)] = (
              acc[myrow + r, pl.ds(col, 16)] + tbuf[r, pl.ds(col, 16)])
      return carry

    lax.fori_loop(1, NS, red_body, 0)

    pltpu.sync_copy(
        acc.at[pl.ds(myrow, ROWS_PER_TILE)],
        out_hbm.at[pl.ds(myrow, ROWS_PER_TILE), pl.ds(col0, DC)])

  return k(feat, ids)


def kernel(feat, segment_ids):
  ids = segment_ids.astype(jnp.int32)
  return _sum_pool(feat, ids)


# speedup vs baseline: 5.6121x; 1.0030x over previous
import functools

import jax
import jax.numpy as jnp
from jax import lax
from jax.experimental import pallas as pl
from jax.experimental.pallas import tpu as pltpu
from jax.experimental.pallas import tpu_sc as plsc

N_ROWS = 50000
D = 256
G = 128
NC = 2
NS = 16
DC = D // NC
BLK = 256
NBLK = N_ROWS // BLK
TAIL = N_ROWS - NBLK * BLK
BASE_BLKS = NBLK // NS
EXTRA_BLKS = NBLK - BASE_BLKS * NS
ROWS_PER_TILE = G // NS
KC = DC // 16


def _sum_pool(feat, ids):
  mesh = plsc.VectorSubcoreMesh(core_axis_name="c", subcore_axis_name="s")

  @functools.partial(
      pl.kernel,
      out_type=jax.ShapeDtypeStruct((G, D), jnp.float32),
      mesh=mesh,
      scratch_types=[
          pltpu.VMEM((2, BLK), jnp.int32),
          pltpu.VMEM((2, BLK, DC), jnp.float32),
          pltpu.VMEM((G, DC), jnp.float32),
          pltpu.VMEM((NS - 1, ROWS_PER_TILE, DC), jnp.float32),
          pltpu.VMEM_SHARED((NS, G, DC), jnp.float32),
          pltpu.SemaphoreType.DMA,
          pltpu.SemaphoreType.DMA,
      ],
  )
  def k(feat_hbm, ids_hbm, out_hbm, ids2, rows2, acc, tbuf, shared, sem0,
        sem1):
    c = lax.axis_index("c")
    s = lax.axis_index("s")
    sems = (sem0, sem1)
    col0 = c * DC

    zero = jnp.zeros((16,), jnp.float32)

    def zero_body(r, carry):
      for kc in range(KC):
        acc[r, pl.ds(kc * 16, 16)] = zero
      return carry

    lax.fori_loop(0, G, zero_body, 0)

    base = s * BASE_BLKS + jnp.minimum(s, EXTRA_BLKS)
    nblk = BASE_BLKS + jnp.where(s < EXTRA_BLKS, 1, 0)

    def accum_group(gbase, idref, rowref):
      idv = idref[pl.ds(gbase, 16)]
      first = idv[0]
      last = idv[15]

      def fast():
        for kc in range(KC):
          col = kc * 16
          v = [rowref[gbase + j, pl.ds(col, 16)] for j in range(16)]
          while len(v) > 1:
            v = [v[i] + v[i + 1] for i in range(0, len(v), 2)]
          acc[first, pl.ds(col, 16)] = acc[first, pl.ds(col, 16)] + v[0]

      def slow():
        for j in range(16):
          sj = idv[j]
          for kc in range(KC):
            col = kc * 16
            acc[sj, pl.ds(col, 16)] = (
                acc[sj, pl.ds(col, 16)] + rowref[gbase + j, pl.ds(col, 16)])

      lax.cond(first == last, fast, slow)

    H = BLK // 2

    def start_blk(g, b):
      r0 = (base + g) * BLK
      pltpu.make_async_copy(
          ids_hbm.at[pl.ds(r0, BLK)], ids2.at[b], sems[b]).start()
      pltpu.make_async_copy(
          feat_hbm.at[pl.ds(r0, H), pl.ds(col0, DC)],
          rows2.at[b, pl.ds(0, H)], sems[b]).start()
      pltpu.make_async_copy(
          feat_hbm.at[pl.ds(r0 + H, H), pl.ds(col0, DC)],
          rows2.at[b, pl.ds(H, H)], sems[b]).start()

    def wait_blk(b):
      pltpu.make_async_copy(
          ids_hbm.at[pl.ds(0, BLK)], ids2.at[b], sems[b]).wait()
      pltpu.make_async_copy(
          feat_hbm.at[pl.ds(0, H), pl.ds(0, DC)],
          rows2.at[b, pl.ds(0, H)], sems[b]).wait()
      pltpu.make_async_copy(
          feat_hbm.at[pl.ds(0, H), pl.ds(0, DC)],
          rows2.at[b, pl.ds(H, H)], sems[b]).wait()

    @pl.when(nblk > 0)
    def _():
      start_blk(0, 0)

    @pl.when(nblk > 1)
    def _():
      start_blk(1, 1)

    def pair_body(p, carry):
      for b in range(2):
        g = p * 2 + b

        @pl.when(g < nblk)
        def _():
          wait_blk(b)

          def group_body(g2, carry2):
            accum_group(g2 * 16, ids2.at[b], rows2.at[b])
            return carry2

          lax.fori_loop(0, BLK // 16, group_body, 0)

          @pl.when(g + 2 < nblk)
          def _():
            start_blk(g + 2, b)

      return carry

    lax.fori_loop(0, (nblk + 1) // 2, pair_body, 0)

    @pl.when(s == NS - 1)
    def _():
      r0 = NBLK * BLK
      pltpu.sync_copy(ids_hbm.at[pl.ds(r0, TAIL)], ids2.at[0, pl.ds(0, TAIL)])
      pltpu.sync_copy(feat_hbm.at[pl.ds(r0, TAIL), pl.ds(col0, DC)],
                      rows2.at[0, pl.ds(0, TAIL)])

      def tail_body(t, carry):
        accum_group(t * 16, ids2.at[0], rows2.at[0])
        return carry

      lax.fori_loop(0, TAIL // 16, tail_body, 0)

    pltpu.sync_copy(acc, shared.at[s])
    plsc.subcore_barrier()

    myrow = s * ROWS_PER_TILE

    def red_start(t, carry):
      pltpu.make_async_copy(
          shared.at[t + 1, pl.ds(myrow, ROWS_PER_TILE)], tbuf.at[t],
          sem0).start()
      return carry

    lax.fori_loop(0, NS - 1, red_start, 0)
    pltpu.sync_copy(
        shared.at[0, pl.ds(myrow, ROWS_PER_TILE)],
        acc.at[pl.ds(myrow, ROWS_PER_TILE)])

    def red_wait(t, carry):
      pltpu.make_async_copy(
          shared.at[0, pl.ds(0, ROWS_PER_TILE)], tbuf.at[t], sem0).wait()
      return carry

    lax.fori_loop(0, NS - 1, red_wait, 0)

    def red_add(t, carry):
      for r in range(ROWS_PER_TILE):
        for kc in range(KC):
          col = kc * 16
          acc[myrow + r, pl.ds(col, 16)] = (
              acc[myrow + r, pl.ds(col, 16)] + tbuf[t, r, pl.ds(col, 16)])
      return carry

    lax.fori_loop(0, NS - 1, red_add, 0)

    pltpu.sync_copy(
        acc.at[pl.ds(myrow, ROWS_PER_TILE)],
        out_hbm.at[pl.ds(myrow, ROWS_PER_TILE), pl.ds(col0, DC)])

  return k(feat, ids)


def kernel(feat, segment_ids):
  ids = segment_ids.astype(jnp.int32)
  return _sum_pool(feat, ids)
